# Initial kernel scaffold; baseline (speedup 1.0000x reference)
import jax
import jax.numpy as jnp
from jax.experimental import pallas as pl


def kernel(ids, length, embedding_table):
    raise NotImplementedError("write your pallas kernel here")


# speedup vs baseline: 1.0548x; 1.0548x over previous
import functools

import jax
import jax.numpy as jnp
from jax import lax
from jax.experimental import pallas as pl
from jax.experimental.pallas import tpu as pltpu
from jax.experimental.pallas import tpu_sc as plsc

_BATCH = 16384
_HIST = 50
_EMBED = 32
_ROWS = _BATCH * _HIST
_NW = 32
_ROWS_W = _ROWS // _NW
_BATCH_W = _BATCH // _NW
_CHUNK = 800
_NCHUNK = _ROWS_W // _CHUNK


@functools.partial(
    pl.kernel,
    mesh=plsc.VectorSubcoreMesh(core_axis_name="c", subcore_axis_name="s"),
    out_type=jax.ShapeDtypeStruct((_ROWS, _EMBED), jnp.float32),
    compiler_params=pltpu.CompilerParams(use_tc_tiling_on_sc=False),
    scratch_types=[
        pltpu.VMEM((_CHUNK,), jnp.int32),
        pltpu.VMEM((_CHUNK, _EMBED), jnp.float32),
        pltpu.VMEM((_BATCH_W + 16,), jnp.int32),
        pltpu.SemaphoreType.DMA,
    ],
)
def _emb_lookup(ids_hbm, len_hbm, table_hbm, out_hbm, idx_v, data_v, len_v, sem):
    wid = lax.axis_index("s") * 2 + lax.axis_index("c")
    row0 = wid * _ROWS_W
    b0 = wid * _BATCH_W
    pltpu.sync_copy(len_hbm.at[pl.ds(b0, _BATCH_W)],
                    len_v.at[pl.ds(0, _BATCH_W)])
    zeros = jnp.zeros((16,), jnp.float32)

    def chunk_body(c, carry):
        base = row0 + c * _CHUNK
        pltpu.sync_copy(ids_hbm.at[pl.ds(base, _CHUNK)], idx_v)
        pltpu.async_copy(table_hbm.at[idx_v], data_v, sem).wait()
        for bi in range(_CHUNK // _HIST):
            ln = len_v[pl.ds(c * (_CHUNK // _HIST) + bi, 16)][0]
            r0 = bi * _HIST

            def zbody(j, zcarry, r0=r0):
                data_v[r0 + j, pl.ds(0, 16)] = zeros
                data_v[r0 + j, pl.ds(16, 16)] = zeros
                return zcarry

            lax.fori_loop(ln, _HIST, zbody, 0)
        pltpu.sync_copy(data_v, out_hbm.at[pl.ds(base, _CHUNK)])
        return carry

    lax.fori_loop(0, _NCHUNK, chunk_body, 0)


def kernel(ids, length, embedding_table):
    out = _emb_lookup(ids.reshape(_ROWS), length, embedding_table)
    return out.reshape(_BATCH, _HIST, _EMBED)


# speedup vs baseline: 1.1007x; 1.0435x over previous
import functools

import jax
import jax.numpy as jnp
from jax import lax
from jax.experimental import pallas as pl
from jax.experimental.pallas import tpu as pltpu
from jax.experimental.pallas import tpu_sc as plsc

_BATCH = 16384
_HIST = 50
_EMBED = 32
_ROWS = _BATCH * _HIST
_NW = 32
_ROWS_W = _ROWS // _NW
_BATCH_W = _BATCH // _NW
_CHUNK = 1600
_CB = _CHUNK // _HIST
_NCHUNK = _ROWS_W // _CHUNK


@functools.partial(
    pl.kernel,
    mesh=plsc.VectorSubcoreMesh(core_axis_name="c", subcore_axis_name="s"),
    out_type=jax.ShapeDtypeStruct((_ROWS, _EMBED), jnp.float32),
    compiler_params=pltpu.CompilerParams(use_tc_tiling_on_sc=False),
    scratch_types=[
        pltpu.VMEM((_CHUNK,), jnp.int32),
        pltpu.VMEM((_CHUNK,), jnp.int32),
        pltpu.VMEM((_CHUNK, _EMBED), jnp.float32),
        pltpu.VMEM((_CHUNK, _EMBED), jnp.float32),
        pltpu.VMEM((_BATCH_W + 16,), jnp.int32),
        pltpu.SemaphoreType.DMA,
        pltpu.SemaphoreType.DMA,
        pltpu.SemaphoreType.DMA,
        pltpu.SemaphoreType.DMA,
    ],
)
def _emb_lookup(ids_hbm, len_hbm, table_hbm, out_hbm,
                idx0, idx1, data0, data1, len_v,
                isem0, isem1, gsem0, gsem1):
    wid = lax.axis_index("s") * 2 + lax.axis_index("c")
    row0 = wid * _ROWS_W
    b0 = wid * _BATCH_W
    pltpu.sync_copy(len_hbm.at[pl.ds(b0, _BATCH_W)],
                    len_v.at[pl.ds(0, _BATCH_W)])
    zeros = jnp.zeros((16,), jnp.float32)

    def ids_slice(c):
        return ids_hbm.at[pl.ds(row0 + c * _CHUNK, _CHUNK)]

    def zero_tails(c, data):
        for bi in range(_CB):
            ln = len_v[pl.ds(c * _CB + bi, 16)][0]
            r0 = bi * _HIST

            def zbody(j, zcarry, r0=r0):
                data[r0 + j, pl.ds(0, 16)] = zeros
                data[r0 + j, pl.ds(16, 16)] = zeros
                return zcarry

            lax.fori_loop(ln, _HIST, zbody, 0)

    def out_copy(c, data):
        pltpu.sync_copy(data, out_hbm.at[pl.ds(row0 + c * _CHUNK, _CHUNK)])

    pltpu.async_copy(ids_slice(0), idx0, isem0)
    pltpu.async_copy(ids_slice(1), idx1, isem1)
    pltpu.make_async_copy(ids_slice(0), idx0, isem0).wait()
    pltpu.async_copy(table_hbm.at[idx0], data0, gsem0)

    def pair_body(i, carry):
        a = 2 * i

        pltpu.make_async_copy(table_hbm.at[idx0], data0, gsem0).wait()

        @pl.when(i < _NCHUNK // 2 - 1)
        def _():
            pltpu.async_copy(ids_slice(a + 2), idx0, isem0)

        pltpu.make_async_copy(ids_slice(a + 1), idx1, isem1).wait()
        pltpu.async_copy(table_hbm.at[idx1], data1, gsem1)
        zero_tails(a, data0)
        out_copy(a, data0)

        pltpu.make_async_copy(table_hbm.at[idx1], data1, gsem1).wait()

        @pl.when(i < _NCHUNK // 2 - 1)
        def _():
            pltpu.async_copy(ids_slice(a + 3), idx1, isem1)
            pltpu.make_async_copy(ids_slice(a + 2), idx0, isem0).wait()
            pltpu.async_copy(table_hbm.at[idx0], data0, gsem0)

        zero_tails(a + 1, data1)
        out_copy(a + 1, data1)
        return carry

    lax.fori_loop(0, _NCHUNK // 2, pair_body, 0)


def kernel(ids, length, embedding_table):
    out = _emb_lookup(ids.reshape(_ROWS), length, embedding_table)
    return out.reshape(_BATCH, _HIST, _EMBED)


# speedup vs baseline: 1.5076x; 1.3696x over previous
import functools

import jax
import jax.numpy as jnp
from jax import lax
from jax.experimental import pallas as pl
from jax.experimental.pallas import tpu as pltpu
from jax.experimental.pallas import tpu_sc as plsc

_BATCH = 16384
_HIST = 50
_EMBED = 32
_NW = 32
_BATCH_W = _BATCH // _NW
_Q = 128
_NQ = _BATCH_W // _Q
_STEPS = _HIST * _NQ
_VOCAB = 1000000
_VW = _VOCAB // 4


@functools.partial(
    pl.kernel,
    mesh=plsc.VectorSubcoreMesh(core_axis_name="c", subcore_axis_name="s"),
    out_type=jax.ShapeDtypeStruct((_HIST, _EMBED, _BATCH), jnp.float32),
    compiler_params=pltpu.CompilerParams(needs_layout_passes=False),
    scratch_types=[
        pltpu.VMEM((_HIST, _BATCH_W), jnp.int32),
        pltpu.VMEM((_BATCH_W,), jnp.int32),
        pltpu.VMEM((_Q,), jnp.int32),
        pltpu.VMEM((_Q,), jnp.int32),
        pltpu.VMEM((_Q,), jnp.int32),
        pltpu.VMEM((_Q,), jnp.int32),
        pltpu.VMEM((_Q, 128), jnp.float32),
        pltpu.VMEM((_Q, 128), jnp.float32),
        pltpu.VMEM((_EMBED, _Q), jnp.float32),
        pltpu.SemaphoreType.DMA,
        pltpu.SemaphoreType.DMA,
    ],
)
def _emb_lookup(idst_hbm, len_hbm, tablew_hbm, outt_hbm,
                idst_v, len_v, idx0, idx1, colb0, colb1, data0, data1,
                stage_v, gsem0, gsem1):
    wid = lax.axis_index("s") * 2 + lax.axis_index("c")
    b0 = wid * _BATCH_W
    pltpu.sync_copy(idst_hbm.at[:, pl.ds(b0, _BATCH_W)], idst_v)
    pltpu.sync_copy(len_hbm.at[pl.ds(b0, _BATCH_W)], len_v)
    iota = lax.iota(jnp.int32, 16)
    idxs = (idx0, idx1)
    colbs = (colb0, colb1)
    datas = (data0, data1)
    gsems = (gsem0, gsem1)

    def build_idx(t, idx_v, colb_v):
        j = t >> 2
        c0 = (t & 3) * _Q
        for ib in range(_Q // 16):
            iv = idst_v[j, pl.ds(c0 + ib * 16, 16)]
            idx_v[pl.ds(ib * 16, 16)] = iv >> 2
            colb_v[pl.ds(ib * 16, 16)] = (iv & 3) << 5

    def start_gather(p):
        pltpu.async_copy(tablew_hbm.at[idxs[p]], datas[p], gsems[p])

    def wait_gather(p):
        pltpu.make_async_copy(tablew_hbm.at[idxs[p]], datas[p],
                              gsems[p]).wait()

    build_idx(0, idx0, colb0)
    start_gather(0)

    def step(t, carry):
        p = t & 1

        @pl.when(t + 1 < _STEPS)
        def _():
            @pl.when(p == 0)
            def _():
                build_idx(t + 1, idx1, colb1)
                start_gather(1)

            @pl.when(p == 1)
            def _():
                build_idx(t + 1, idx0, colb0)
                start_gather(0)

        j = t >> 2
        c0 = (t & 3) * _Q

        def consume(p_static):
            wait_gather(p_static)
            dv = datas[p_static]
            cbv = colbs[p_static]
            for ib in range(_Q // 16):
                rows = iota + ib * 16
                lenv = len_v[pl.ds(c0 + ib * 16, 16)]
                keep = lenv > j
                colb = cbv[pl.ds(ib * 16, 16)]
                for d in range(_EMBED):
                    g = plsc.load_gather(dv, [rows, colb + d])
                    stage_v[d, pl.ds(ib * 16, 16)] = jnp.where(keep, g, 0.0)

        @pl.when(p == 0)
        def _():
            consume(0)

        @pl.when(p == 1)
        def _():
            consume(1)

        pltpu.sync_copy(stage_v, outt_hbm.at[j, :, pl.ds(b0 + c0, _Q)])
        return carry

    lax.fori_loop(0, _STEPS, step, 0)


def kernel(ids, length, embedding_table):
    out_t = _emb_lookup(ids.T, length, embedding_table.reshape(_VW, 128))
    return out_t.transpose(2, 0, 1)
